# baseline (speedup 1.0000x reference)
import jax
import jax.numpy as jnp
import numpy as np
from jax.experimental import pallas as pl
from jax.experimental.pallas import tpu as pltpu

N = 2048
D = 512
BW = 512
K1 = 25
RB = 256
INV_SQRT_D = np.float32(1.0 / (512.0 ** 0.5))
IMIN = np.int32(-2147483648)
IFLIP = np.int32(2147483647)
TWO_PI_3 = np.float32(2.0943951023931953)


def _key_of(x):
    b = jax.lax.bitcast_convert_type(x, jnp.int32)
    return jnp.where(b < 0, IFLIP ^ b, b)


def _val_of(k):
    return jax.lax.bitcast_convert_type(jnp.where(k < 0, IFLIP ^ k, k),
                                        jnp.float32)


def _two_sum(a, b):
    s = a + b
    bb = s - a
    err = (a - (s - bb)) + (b - bb)
    return s, err


def _split(a):
    t = a * 4097.0
    hi = t - (t - a)
    return hi, a - hi


def _two_prod(a, b):
    p = a * b
    ah, al = _split(a)
    bh, bl = _split(b)
    err = ((ah * bh - p) + ah * bl + al * bh) + al * bl
    return p, err


def _df_add(ah, al, bh, bl):
    s, e = _two_sum(ah, bh)
    e = e + (al + bl)
    hi, lo = _two_sum(s, e)
    return hi, lo


def _df_mul_f(ah, al, b):
    p, e = _two_prod(ah, b)
    e = e + al * b
    hi, lo = _two_sum(p, e)
    return hi, lo


def _poly_coeffs_df(a, b, c, d, e, f):
    th, tl = _two_sum(a, d)
    th, tl2 = _two_sum(th, f)
    tl = tl + tl2

    def minor(p, q, r):
        m1h, m1l = _two_prod(p, q)
        m2h, m2l = _two_prod(r, r)
        return _df_add(m1h, m1l, -m2h, -m2l)

    m1h, m1l = minor(a, d, b)
    m2h, m2l = minor(a, f, c)
    m3h, m3l = minor(d, f, e)
    mh, ml = _df_add(m1h, m1l, m2h, m2l)
    mh, ml = _df_add(mh, ml, m3h, m3l)
    d1h, d1l = minor(d, f, e)
    d1h, d1l = _df_mul_f(d1h, d1l, a)
    bfh, bfl = _two_prod(b, f)
    ech, ecl = _two_prod(e, c)
    t2h, t2l = _df_add(bfh, bfl, -ech, -ecl)
    t2h, t2l = _df_mul_f(t2h, t2l, b)
    beh, bel = _two_prod(b, e)
    dch, dcl = _two_prod(d, c)
    t3h, t3l = _df_add(beh, bel, -dch, -dcl)
    t3h, t3l = _df_mul_f(t3h, t3l, c)
    dh, dl = _df_add(d1h, d1l, -t2h, -t2l)
    dh, dl = _df_add(dh, dl, t3h, t3l)
    return (th, tl), (mh, ml), (dh, dl)


def _newton(s, T, M, Det, scale):
    th, tl = T
    mh, ml = M
    dh, dl = Det
    for _ in range(2):
        hh, hl = _df_add(s, jnp.zeros_like(s), -th, -tl)
        hh, hl = _df_mul_f(hh, hl, s)
        hh, hl = _df_add(hh, hl, mh, ml)
        hh, hl = _df_mul_f(hh, hl, s)
        hh, hl = _df_add(hh, hl, -dh, -dl)
        p = hh + hl
        dp = (3.0 * s - 2.0 * th) * s + mh
        ok = jnp.abs(dp) > 1e-10 * scale * scale
        u = jnp.where(ok, -p / jnp.where(ok, dp, 1.0), 0.0)
        u = jnp.clip(u, -scale, scale)
        s = s + u
    return s


def _newton_f32(s, th, mh, dh, n):
    for _ in range(n):
        p = ((s - th) * s + mh) * s - dh
        dp = (3.0 * s - 2.0 * th) * s + mh
        ok = jnp.abs(dp) > 0.0
        s = s - jnp.where(ok, p / jnp.where(ok, dp, 1.0), 0.0)
    return s


def _eig_e(a, b, c, d, e, f):
    q = (a + d + f) / 3.0
    p1 = b * b + c * c + e * e
    p2 = ((a - q) * (a - q) + (d - q) * (d - q) + (f - q) * (f - q)
          + 2.0 * p1)
    p = jnp.sqrt(p2 / 6.0)
    T, M, Det = _poly_coeffs_df(a, b, c, d, e, f)
    th, tl = T
    mh, _ = M
    dh, _ = Det
    scale = jnp.abs(q) + 2.0 * p + 1e-30
    lmax = _newton_f32(q + 2.0 * p, th, mh, dh, 10)
    lmin = _newton_f32(q - 2.0 * p, th, mh, dh, 10)
    lmax = _newton(lmax, T, M, Det, scale)
    lmin = _newton(lmin, T, M, Det, scale)
    midh, midl = _df_add(th, tl, -lmax, jnp.zeros_like(lmax))
    midh, midl = _df_add(midh, midl, -lmin, jnp.zeros_like(lmin))
    lmid = midh + midl
    lmid = _newton(lmid, T, M, Det, scale)
    lmax_f = jnp.maximum(lmax, lmid)
    lmid_f = jnp.maximum(jnp.minimum(lmax, lmid), lmin)
    return 1.0 - lmid_f / (lmax_f + 1e-12)


def _sum25(v):
    acc = ((v[:, 0:8] + v[:, 8:16]) + v[:, 16:24]) + v[:, 24:32]
    f = acc[:, 0:4] + acc[:, 4:8]
    f = f[:, 0:2] + f[:, 2:4]
    return f[:, 0:1] + f[:, 1:2]


def _aniso_kernel(x_ref, y_ref, z_ref, xt_ref, yt_ref, zt_ref, p_ref, pt_ref,
                  e_ref):
    x, y, z = x_ref[...], y_ref[...], z_ref[...]
    xt, yt, zt = xt_ref[...], yt_ref[...], zt_ref[...]
    dot = jnp.dot(p_ref[...], pt_ref[...], preferred_element_type=jnp.float32)
    sq_i = (x * x + z * z) + y * y
    sq_j = (xt * xt + zt * zt) + yt * yt
    negd2 = -((sq_i + sq_j) - 2.0 * dot)

    keys = _key_of(negd2)
    lane = jax.lax.broadcasted_iota(jnp.int32, (RB, N), 1)
    lane128 = jax.lax.broadcasted_iota(jnp.int32, (RB, 128), 1)
    xb = jnp.broadcast_to(xt, (RB, N))
    yb = jnp.broadcast_to(yt, (RB, N))
    zb = jnp.broadcast_to(zt, (RB, N))
    ninf = jnp.float32(-jnp.inf)
    nbx = jnp.zeros((RB, 128), jnp.float32)
    nby = jnp.zeros((RB, 128), jnp.float32)
    nbz = jnp.zeros((RB, 128), jnp.float32)
    for t in range(K1):
        m = jnp.max(keys, axis=1, keepdims=True)
        sel = keys == m
        idx = jnp.min(jnp.where(sel, lane, N), axis=1, keepdims=True)
        one = lane == idx
        px = jnp.max(jnp.where(one, xb, ninf), axis=1, keepdims=True)
        py = jnp.max(jnp.where(one, yb, ninf), axis=1, keepdims=True)
        pz = jnp.max(jnp.where(one, zb, ninf), axis=1, keepdims=True)
        tm = lane128 == t
        nbx = jnp.where(tm, px, nbx)
        nby = jnp.where(tm, py, nby)
        nbz = jnp.where(tm, pz, nbz)
        keys = jnp.where(one, IMIN, keys)

    s04 = np.float32(0.04)
    mux = _sum25(nbx) * s04
    muy = _sum25(nby) * s04
    muz = _sum25(nbz) * s04
    valid = lane128 < K1
    cx = jnp.where(valid, nbx - mux, 0.0)
    cy = jnp.where(valid, nby - muy, 0.0)
    cz = jnp.where(valid, nbz - muz, 0.0)
    cx = cx.astype(jnp.bfloat16).astype(jnp.float32)
    cy = cy.astype(jnp.bfloat16).astype(jnp.float32)
    cz = cz.astype(jnp.bfloat16).astype(jnp.float32)
    cxx = jnp.sum(cx * cx, axis=1, keepdims=True) * s04
    cxy = jnp.sum(cx * cy, axis=1, keepdims=True) * s04
    cxz = jnp.sum(cx * cz, axis=1, keepdims=True) * s04
    cyy = jnp.sum(cy * cy, axis=1, keepdims=True) * s04
    cyz = jnp.sum(cy * cz, axis=1, keepdims=True) * s04
    czz = jnp.sum(cz * cz, axis=1, keepdims=True) * s04
    del s04
    e_ref[...] = _eig_e(cxx, cxy, cxz, cyy, cyz, czz)


def _aniso(pts):
    x = pts[:, 0:1]
    y = pts[:, 1:2]
    z = pts[:, 2:3]
    col = pl.BlockSpec((RB, 1), lambda i: (i, 0))
    row = pl.BlockSpec((1, N), lambda i: (0, 0))
    return pl.pallas_call(
        _aniso_kernel,
        grid=(N // RB,),
        in_specs=[col, col, col, row, row, row,
                  pl.BlockSpec((RB, 3), lambda i: (i, 0)),
                  pl.BlockSpec((3, N), lambda i: (0, 0))],
        out_specs=col,
        out_shape=jax.ShapeDtypeStruct((N, 1), jnp.float32),
    )(x, y, z, x.T, y.T, z.T, pts, pts.T)


def _aff_kernel(a_ref, bt_ref, aff_ref, afft_ref):
    aff = jnp.dot(a_ref[...], bt_ref[...],
                  preferred_element_type=jnp.float32) * INV_SQRT_D
    aff_ref[...] = aff
    afft_ref[...] = aff.T


def _aff(s_feat, r_featT):
    return pl.pallas_call(
        _aff_kernel,
        grid=(N // RB,),
        in_specs=[pl.BlockSpec((RB, D), lambda i: (i, 0)),
                  pl.BlockSpec((D, N), lambda i: (0, 0))],
        out_specs=(pl.BlockSpec((RB, N), lambda i: (i, 0)),
                   pl.BlockSpec((N, RB), lambda i: (0, i))),
        out_shape=(jax.ShapeDtypeStruct((N, N), jnp.float32),
                   jax.ShapeDtypeStruct((N, N), jnp.float32)),
    )(s_feat, r_featT)


def _colstats_kernel(a_ref, mx_ref, sm_ref):
    a = a_ref[...]
    mx = jnp.max(a, axis=0, keepdims=True)
    ex = jnp.exp(a - mx)
    sm_ref[...] = jnp.sum(ex, axis=0, keepdims=True)
    mx_ref[...] = mx


def _colstats(aff):
    CB = 256
    return pl.pallas_call(
        _colstats_kernel,
        grid=(N // CB,),
        in_specs=[pl.BlockSpec((N, CB), lambda i: (0, i))],
        out_specs=(pl.BlockSpec((1, CB), lambda i: (0, i)),
                   pl.BlockSpec((1, CB), lambda i: (0, i))),
        out_shape=(jax.ShapeDtypeStruct((1, N), jnp.float32),
                   jax.ShapeDtypeStruct((1, N), jnp.float32)),
    )(aff)


def _score_kernel(a_ref, cmx_ref, csm_ref, rmx_ref, rsm_ref, es_ref, er_ref,
                  k_ref):
    a = a_ref[...]
    e0 = jnp.exp(a - cmx_ref[...])
    d13 = e0 / csm_ref[...]
    e1 = jnp.exp(a - rmx_ref[...])
    d12 = e1 / rsm_ref[...]
    sc = ((d13 * d12) * es_ref[...]) * er_ref[...]
    k_ref[...] = _key_of(sc)


def _scores(aff, cmx, csm, rmx, rsm, e_src, e_ref):
    col = pl.BlockSpec((RB, 1), lambda i: (i, 0))
    row = pl.BlockSpec((1, N), lambda i: (0, 0))
    return pl.pallas_call(
        _score_kernel,
        grid=(N // RB,),
        in_specs=[pl.BlockSpec((RB, N), lambda i: (i, 0)),
                  row, row, col, col, col, row],
        out_specs=pl.BlockSpec((RB, N), lambda i: (i, 0)),
        out_shape=jax.ShapeDtypeStruct((N, N), jnp.int32),
    )(aff, cmx, csm, rmx, rsm, e_src, e_ref)


def _topk_kernel(k_ref, v_ref, s_ref, r_ref, keys, rmax):
    keys[...] = k_ref[...]
    rmax[...] = jnp.max(k_ref[...], axis=1, keepdims=True)
    rio = jax.lax.broadcasted_iota(jnp.int32, (N, 1), 0)
    cio = jax.lax.broadcasted_iota(jnp.int32, (1, N), 1)
    tio = jax.lax.broadcasted_iota(jnp.int32, (1, BW), 1)

    def body(t, carry):
        vacc, sacc, racc = carry
        rm = rmax[...]
        m = jnp.max(rm)
        r = jnp.min(jnp.where(rm == m, rio, N))
        rowk = keys[pl.ds(r, 1), :]
        c = jnp.min(jnp.where(rowk == m, cio, N))
        here = tio == t
        vacc = jnp.where(here, _val_of(m), vacc)
        sacc = jnp.where(here, r, sacc)
        racc = jnp.where(here, c, racc)
        newrow = jnp.where(cio == c, IMIN, rowk)
        keys[pl.ds(r, 1), :] = newrow
        rmax[pl.ds(r, 1), :] = jnp.max(newrow, axis=1, keepdims=True)
        return vacc, sacc, racc

    v0 = jnp.zeros((1, BW), jnp.float32)
    i0 = jnp.zeros((1, BW), jnp.int32)
    v, s, r = jax.lax.fori_loop(0, BW, body, (v0, i0, i0))
    v_ref[...] = v
    s_ref[...] = s
    r_ref[...] = r


def _topk(keysmat):
    return pl.pallas_call(
        _topk_kernel,
        out_shape=(jax.ShapeDtypeStruct((1, BW), jnp.float32),
                   jax.ShapeDtypeStruct((1, BW), jnp.int32),
                   jax.ShapeDtypeStruct((1, BW), jnp.int32)),
        scratch_shapes=[pltpu.VMEM((N, N), jnp.int32),
                        pltpu.VMEM((N, 1), jnp.int32)],
    )(keysmat)


def kernel(src_points_f, ref_points_f, s_n_features, r_n_features,
           gt_transform, src_points, ref_points):
    e_src = _aniso(src_points_f)
    e_ref = _aniso(ref_points_f)
    aff, afft = _aff(s_n_features, r_n_features.T)
    cmx, csm = _colstats(aff)
    rmx_t, rsm_t = _colstats(afft)
    keys = _scores(aff, cmx, csm, rmx_t.T, rsm_t.T, e_src, e_ref.T)
    vals, src, ref = _topk(keys)
    return vals.reshape(BW), src.reshape(BW), ref.reshape(BW)


# speedup vs baseline: 1.1126x; 1.1126x over previous
import jax
import jax.numpy as jnp
import numpy as np
from jax.experimental import pallas as pl
from jax.experimental.pallas import tpu as pltpu

N = 2048
D = 512
BW = 512
K1 = 25
RB = 256
INV_SQRT_D = np.float32(1.0 / (512.0 ** 0.5))
IMIN = np.int32(-2147483648)
IFLIP = np.int32(2147483647)
TWO_PI_3 = np.float32(2.0943951023931953)


def _key_of(x):
    b = jax.lax.bitcast_convert_type(x, jnp.int32)
    return jnp.where(b < 0, IFLIP ^ b, b)


def _val_of(k):
    return jax.lax.bitcast_convert_type(jnp.where(k < 0, IFLIP ^ k, k),
                                        jnp.float32)


def _two_sum(a, b):
    s = a + b
    bb = s - a
    err = (a - (s - bb)) + (b - bb)
    return s, err


def _split(a):
    t = a * 4097.0
    hi = t - (t - a)
    return hi, a - hi


def _two_prod(a, b):
    p = a * b
    ah, al = _split(a)
    bh, bl = _split(b)
    err = ((ah * bh - p) + ah * bl + al * bh) + al * bl
    return p, err


def _df_add(ah, al, bh, bl):
    s, e = _two_sum(ah, bh)
    e = e + (al + bl)
    hi, lo = _two_sum(s, e)
    return hi, lo


def _df_mul_f(ah, al, b):
    p, e = _two_prod(ah, b)
    e = e + al * b
    hi, lo = _two_sum(p, e)
    return hi, lo


def _poly_coeffs_df(a, b, c, d, e, f):
    th, tl = _two_sum(a, d)
    th, tl2 = _two_sum(th, f)
    tl = tl + tl2

    def minor(p, q, r):
        m1h, m1l = _two_prod(p, q)
        m2h, m2l = _two_prod(r, r)
        return _df_add(m1h, m1l, -m2h, -m2l)

    m1h, m1l = minor(a, d, b)
    m2h, m2l = minor(a, f, c)
    m3h, m3l = minor(d, f, e)
    mh, ml = _df_add(m1h, m1l, m2h, m2l)
    mh, ml = _df_add(mh, ml, m3h, m3l)
    d1h, d1l = minor(d, f, e)
    d1h, d1l = _df_mul_f(d1h, d1l, a)
    bfh, bfl = _two_prod(b, f)
    ech, ecl = _two_prod(e, c)
    t2h, t2l = _df_add(bfh, bfl, -ech, -ecl)
    t2h, t2l = _df_mul_f(t2h, t2l, b)
    beh, bel = _two_prod(b, e)
    dch, dcl = _two_prod(d, c)
    t3h, t3l = _df_add(beh, bel, -dch, -dcl)
    t3h, t3l = _df_mul_f(t3h, t3l, c)
    dh, dl = _df_add(d1h, d1l, -t2h, -t2l)
    dh, dl = _df_add(dh, dl, t3h, t3l)
    return (th, tl), (mh, ml), (dh, dl)


def _newton(s, T, M, Det, scale):
    th, tl = T
    mh, ml = M
    dh, dl = Det
    for _ in range(2):
        hh, hl = _df_add(s, jnp.zeros_like(s), -th, -tl)
        hh, hl = _df_mul_f(hh, hl, s)
        hh, hl = _df_add(hh, hl, mh, ml)
        hh, hl = _df_mul_f(hh, hl, s)
        hh, hl = _df_add(hh, hl, -dh, -dl)
        p = hh + hl
        dp = (3.0 * s - 2.0 * th) * s + mh
        ok = jnp.abs(dp) > 1e-10 * scale * scale
        u = jnp.where(ok, -p / jnp.where(ok, dp, 1.0), 0.0)
        u = jnp.clip(u, -scale, scale)
        s = s + u
    return s


def _newton_f32(s, th, mh, dh, n):
    for _ in range(n):
        p = ((s - th) * s + mh) * s - dh
        dp = (3.0 * s - 2.0 * th) * s + mh
        ok = jnp.abs(dp) > 0.0
        s = s - jnp.where(ok, p / jnp.where(ok, dp, 1.0), 0.0)
    return s


def _eig_e(a, b, c, d, e, f):
    q = (a + d + f) / 3.0
    p1 = b * b + c * c + e * e
    p2 = ((a - q) * (a - q) + (d - q) * (d - q) + (f - q) * (f - q)
          + 2.0 * p1)
    p = jnp.sqrt(p2 / 6.0)
    T, M, Det = _poly_coeffs_df(a, b, c, d, e, f)
    th, tl = T
    mh, _ = M
    dh, _ = Det
    scale = jnp.abs(q) + 2.0 * p + 1e-30
    lmax = _newton_f32(q + 2.0 * p, th, mh, dh, 10)
    lmin = _newton_f32(q - 2.0 * p, th, mh, dh, 10)
    lmax = _newton(lmax, T, M, Det, scale)
    lmin = _newton(lmin, T, M, Det, scale)
    midh, midl = _df_add(th, tl, -lmax, jnp.zeros_like(lmax))
    midh, midl = _df_add(midh, midl, -lmin, jnp.zeros_like(lmin))
    lmid = midh + midl
    lmid = _newton(lmid, T, M, Det, scale)
    lmax_f = jnp.maximum(lmax, lmid)
    lmid_f = jnp.maximum(jnp.minimum(lmax, lmid), lmin)
    return 1.0 - lmid_f / (lmax_f + 1e-12)


def _sum25(v):
    acc = ((v[:, 0:8] + v[:, 8:16]) + v[:, 16:24]) + v[:, 24:32]
    f = acc[:, 0:4] + acc[:, 4:8]
    f = f[:, 0:2] + f[:, 2:4]
    return f[:, 0:1] + f[:, 1:2]


def _aniso_kernel(x_ref, y_ref, z_ref, xt_ref, yt_ref, zt_ref, p_ref, pt_ref,
                  e_ref):
    x, y, z = x_ref[...], y_ref[...], z_ref[...]
    xt, yt, zt = xt_ref[...], yt_ref[...], zt_ref[...]
    dot = jnp.dot(p_ref[...], pt_ref[...], preferred_element_type=jnp.float32)
    sq_i = (x * x + z * z) + y * y
    sq_j = (xt * xt + zt * zt) + yt * yt
    negd2 = -((sq_i + sq_j) - 2.0 * dot)

    keys = _key_of(negd2)
    lane = jax.lax.broadcasted_iota(jnp.int32, (RB, N), 1)
    lane128 = jax.lax.broadcasted_iota(jnp.int32, (RB, 128), 1)
    xb = jnp.broadcast_to(xt, (RB, N))
    yb = jnp.broadcast_to(yt, (RB, N))
    zb = jnp.broadcast_to(zt, (RB, N))
    ninf = jnp.float32(-jnp.inf)
    nbx = jnp.zeros((RB, 128), jnp.float32)
    nby = jnp.zeros((RB, 128), jnp.float32)
    nbz = jnp.zeros((RB, 128), jnp.float32)
    for t in range(K1):
        m = jnp.max(keys, axis=1, keepdims=True)
        sel = keys == m
        idx = jnp.min(jnp.where(sel, lane, N), axis=1, keepdims=True)
        one = lane == idx
        px = jnp.max(jnp.where(one, xb, ninf), axis=1, keepdims=True)
        py = jnp.max(jnp.where(one, yb, ninf), axis=1, keepdims=True)
        pz = jnp.max(jnp.where(one, zb, ninf), axis=1, keepdims=True)
        tm = lane128 == t
        nbx = jnp.where(tm, px, nbx)
        nby = jnp.where(tm, py, nby)
        nbz = jnp.where(tm, pz, nbz)
        keys = jnp.where(one, IMIN, keys)

    s04 = np.float32(0.04)
    mux = _sum25(nbx) * s04
    muy = _sum25(nby) * s04
    muz = _sum25(nbz) * s04
    valid = lane128 < K1
    cx = jnp.where(valid, nbx - mux, 0.0)
    cy = jnp.where(valid, nby - muy, 0.0)
    cz = jnp.where(valid, nbz - muz, 0.0)
    cx = cx.astype(jnp.bfloat16).astype(jnp.float32)
    cy = cy.astype(jnp.bfloat16).astype(jnp.float32)
    cz = cz.astype(jnp.bfloat16).astype(jnp.float32)
    cxx = jnp.sum(cx * cx, axis=1, keepdims=True) * s04
    cxy = jnp.sum(cx * cy, axis=1, keepdims=True) * s04
    cxz = jnp.sum(cx * cz, axis=1, keepdims=True) * s04
    cyy = jnp.sum(cy * cy, axis=1, keepdims=True) * s04
    cyz = jnp.sum(cy * cz, axis=1, keepdims=True) * s04
    czz = jnp.sum(cz * cz, axis=1, keepdims=True) * s04
    del s04
    e_ref[...] = _eig_e(cxx, cxy, cxz, cyy, cyz, czz)


def _aniso(pts):
    x = pts[:, 0:1]
    y = pts[:, 1:2]
    z = pts[:, 2:3]
    col = pl.BlockSpec((RB, 1), lambda i: (i, 0))
    row = pl.BlockSpec((1, N), lambda i: (0, 0))
    return pl.pallas_call(
        _aniso_kernel,
        grid=(N // RB,),
        in_specs=[col, col, col, row, row, row,
                  pl.BlockSpec((RB, 3), lambda i: (i, 0)),
                  pl.BlockSpec((3, N), lambda i: (0, 0))],
        out_specs=col,
        out_shape=jax.ShapeDtypeStruct((N, 1), jnp.float32),
    )(x, y, z, x.T, y.T, z.T, pts, pts.T)


def _aff_kernel(a_ref, bt_ref, aff_ref, afft_ref):
    aff = jnp.dot(a_ref[...], bt_ref[...],
                  preferred_element_type=jnp.float32) * INV_SQRT_D
    aff_ref[...] = aff
    afft_ref[...] = aff.T


def _aff(s_feat, r_featT):
    return pl.pallas_call(
        _aff_kernel,
        grid=(N // RB,),
        in_specs=[pl.BlockSpec((RB, D), lambda i: (i, 0)),
                  pl.BlockSpec((D, N), lambda i: (0, 0))],
        out_specs=(pl.BlockSpec((RB, N), lambda i: (i, 0)),
                   pl.BlockSpec((N, RB), lambda i: (0, i))),
        out_shape=(jax.ShapeDtypeStruct((N, N), jnp.float32),
                   jax.ShapeDtypeStruct((N, N), jnp.float32)),
    )(s_feat, r_featT)


def _colstats_kernel(a_ref, mx_ref, sm_ref):
    a = a_ref[...]
    mx = jnp.max(a, axis=0, keepdims=True)
    ex = jnp.exp(a - mx)
    sm_ref[...] = jnp.sum(ex, axis=0, keepdims=True)
    mx_ref[...] = mx


def _colstats(aff):
    CB = 256
    return pl.pallas_call(
        _colstats_kernel,
        grid=(N // CB,),
        in_specs=[pl.BlockSpec((N, CB), lambda i: (0, i))],
        out_specs=(pl.BlockSpec((1, CB), lambda i: (0, i)),
                   pl.BlockSpec((1, CB), lambda i: (0, i))),
        out_shape=(jax.ShapeDtypeStruct((1, N), jnp.float32),
                   jax.ShapeDtypeStruct((1, N), jnp.float32)),
    )(aff)


def _score_kernel(a_ref, cmx_ref, csm_ref, rmx_ref, rsm_ref, es_ref, er_ref,
                  k_ref, rm_ref):
    a = a_ref[...]
    e0 = jnp.exp(a - cmx_ref[...])
    d13 = e0 / csm_ref[...]
    e1 = jnp.exp(a - rmx_ref[...])
    d12 = e1 / rsm_ref[...]
    sc = ((d13 * d12) * es_ref[...]) * er_ref[...]
    k = _key_of(sc)
    k_ref[...] = k
    rm_ref[...] = jnp.max(k, axis=1, keepdims=True)


def _scores(aff, cmx, csm, rmx, rsm, e_src, e_ref):
    col = pl.BlockSpec((RB, 1), lambda i: (i, 0))
    row = pl.BlockSpec((1, N), lambda i: (0, 0))
    return pl.pallas_call(
        _score_kernel,
        grid=(N // RB,),
        in_specs=[pl.BlockSpec((RB, N), lambda i: (i, 0)),
                  row, row, col, col, col, row],
        out_specs=(pl.BlockSpec((RB, N), lambda i: (i, 0)),
                   pl.BlockSpec((RB, 1), lambda i: (i, 0))),
        out_shape=(jax.ShapeDtypeStruct((N, N), jnp.int32),
                   jax.ShapeDtypeStruct((N, 1), jnp.int32)),
    )(aff, cmx, csm, rmx, rsm, e_src, e_ref)


def _topk_kernel(k_ref, rm_ref, v_ref, s_ref, r_ref, keys):
    keys[...] = k_ref[...]
    cio = jax.lax.broadcasted_iota(jnp.int32, (1, N), 1)
    tio = jax.lax.broadcasted_iota(jnp.int32, (1, BW), 1)
    fio = (jax.lax.broadcasted_iota(jnp.int32, (16, 128), 0) * 128
           + jax.lax.broadcasted_iota(jnp.int32, (16, 128), 1))
    rm0 = rm_ref[...]

    def body(t, carry):
        vacc, sacc, racc, rm = carry
        m = jnp.max(rm)
        r = jnp.min(jnp.where(rm == m, fio, N))
        rowk = keys[pl.ds(r, 1), :]
        c = jnp.min(jnp.where(rowk == m, cio, N))
        here = tio == t
        vacc = jnp.where(here, _val_of(m), vacc)
        sacc = jnp.where(here, r, sacc)
        racc = jnp.where(here, c, racc)
        newrow = jnp.where(cio == c, IMIN, rowk)
        keys[pl.ds(r, 1), :] = newrow
        rm = jnp.where(fio == r, jnp.max(newrow), rm)
        return vacc, sacc, racc, rm

    v0 = jnp.zeros((1, BW), jnp.float32)
    i0 = jnp.zeros((1, BW), jnp.int32)
    v, s, r, _ = jax.lax.fori_loop(0, BW, body, (v0, i0, i0, rm0))
    v_ref[...] = v
    s_ref[...] = s
    r_ref[...] = r


def _topk(keysmat, rowmax16):
    return pl.pallas_call(
        _topk_kernel,
        out_shape=(jax.ShapeDtypeStruct((1, BW), jnp.float32),
                   jax.ShapeDtypeStruct((1, BW), jnp.int32),
                   jax.ShapeDtypeStruct((1, BW), jnp.int32)),
        scratch_shapes=[pltpu.VMEM((N, N), jnp.int32)],
    )(keysmat, rowmax16)


def kernel(src_points_f, ref_points_f, s_n_features, r_n_features,
           gt_transform, src_points, ref_points):
    e_src = _aniso(src_points_f)
    e_ref = _aniso(ref_points_f)
    aff, afft = _aff(s_n_features, r_n_features.T)
    cmx, csm = _colstats(aff)
    rmx_t, rsm_t = _colstats(afft)
    keys, rowmax = _scores(aff, cmx, csm, rmx_t.T, rsm_t.T, e_src, e_ref.T)
    vals, src, ref = _topk(keys, rowmax.reshape(16, 128))
    return vals.reshape(BW), src.reshape(BW), ref.reshape(BW)


# speedup vs baseline: 1.1127x; 1.0000x over previous
import jax
import jax.numpy as jnp
import numpy as np
from jax.experimental import pallas as pl
from jax.experimental.pallas import tpu as pltpu

N = 2048
D = 512
BW = 512
K1 = 25
RB = 256
INV_SQRT_D = np.float32(1.0 / (512.0 ** 0.5))
IMIN = np.int32(-2147483648)
IFLIP = np.int32(2147483647)
TWO_PI_3 = np.float32(2.0943951023931953)


def _key_of(x):
    b = jax.lax.bitcast_convert_type(x, jnp.int32)
    return jnp.where(b < 0, IFLIP ^ b, b)


def _val_of(k):
    return jax.lax.bitcast_convert_type(jnp.where(k < 0, IFLIP ^ k, k),
                                        jnp.float32)


def _two_sum(a, b):
    s = a + b
    bb = s - a
    err = (a - (s - bb)) + (b - bb)
    return s, err


def _split(a):
    t = a * 4097.0
    hi = t - (t - a)
    return hi, a - hi


def _two_prod(a, b):
    p = a * b
    ah, al = _split(a)
    bh, bl = _split(b)
    err = ((ah * bh - p) + ah * bl + al * bh) + al * bl
    return p, err


def _df_add(ah, al, bh, bl):
    s, e = _two_sum(ah, bh)
    e = e + (al + bl)
    hi, lo = _two_sum(s, e)
    return hi, lo


def _df_mul_f(ah, al, b):
    p, e = _two_prod(ah, b)
    e = e + al * b
    hi, lo = _two_sum(p, e)
    return hi, lo


def _poly_coeffs_df(a, b, c, d, e, f):
    th, tl = _two_sum(a, d)
    th, tl2 = _two_sum(th, f)
    tl = tl + tl2

    def minor(p, q, r):
        m1h, m1l = _two_prod(p, q)
        m2h, m2l = _two_prod(r, r)
        return _df_add(m1h, m1l, -m2h, -m2l)

    m1h, m1l = minor(a, d, b)
    m2h, m2l = minor(a, f, c)
    m3h, m3l = minor(d, f, e)
    mh, ml = _df_add(m1h, m1l, m2h, m2l)
    mh, ml = _df_add(mh, ml, m3h, m3l)
    d1h, d1l = minor(d, f, e)
    d1h, d1l = _df_mul_f(d1h, d1l, a)
    bfh, bfl = _two_prod(b, f)
    ech, ecl = _two_prod(e, c)
    t2h, t2l = _df_add(bfh, bfl, -ech, -ecl)
    t2h, t2l = _df_mul_f(t2h, t2l, b)
    beh, bel = _two_prod(b, e)
    dch, dcl = _two_prod(d, c)
    t3h, t3l = _df_add(beh, bel, -dch, -dcl)
    t3h, t3l = _df_mul_f(t3h, t3l, c)
    dh, dl = _df_add(d1h, d1l, -t2h, -t2l)
    dh, dl = _df_add(dh, dl, t3h, t3l)
    return (th, tl), (mh, ml), (dh, dl)


def _newton(s, T, M, Det, scale):
    th, tl = T
    mh, ml = M
    dh, dl = Det
    for _ in range(2):
        hh, hl = _df_add(s, jnp.zeros_like(s), -th, -tl)
        hh, hl = _df_mul_f(hh, hl, s)
        hh, hl = _df_add(hh, hl, mh, ml)
        hh, hl = _df_mul_f(hh, hl, s)
        hh, hl = _df_add(hh, hl, -dh, -dl)
        p = hh + hl
        dp = (3.0 * s - 2.0 * th) * s + mh
        ok = jnp.abs(dp) > 1e-10 * scale * scale
        u = jnp.where(ok, -p / jnp.where(ok, dp, 1.0), 0.0)
        u = jnp.clip(u, -scale, scale)
        s = s + u
    return s


def _newton_f32(s, th, mh, dh, n):
    for _ in range(n):
        p = ((s - th) * s + mh) * s - dh
        dp = (3.0 * s - 2.0 * th) * s + mh
        ok = jnp.abs(dp) > 0.0
        s = s - jnp.where(ok, p / jnp.where(ok, dp, 1.0), 0.0)
    return s


def _eig_e(a, b, c, d, e, f):
    q = (a + d + f) / 3.0
    p1 = b * b + c * c + e * e
    p2 = ((a - q) * (a - q) + (d - q) * (d - q) + (f - q) * (f - q)
          + 2.0 * p1)
    p = jnp.sqrt(p2 / 6.0)
    T, M, Det = _poly_coeffs_df(a, b, c, d, e, f)
    th, tl = T
    mh, _ = M
    dh, _ = Det
    scale = jnp.abs(q) + 2.0 * p + 1e-30
    lmax = _newton_f32(q + 2.0 * p, th, mh, dh, 10)
    lmin = _newton_f32(q - 2.0 * p, th, mh, dh, 10)
    lmax = _newton(lmax, T, M, Det, scale)
    lmin = _newton(lmin, T, M, Det, scale)
    midh, midl = _df_add(th, tl, -lmax, jnp.zeros_like(lmax))
    midh, midl = _df_add(midh, midl, -lmin, jnp.zeros_like(lmin))
    lmid = midh + midl
    lmid = _newton(lmid, T, M, Det, scale)
    lmax_f = jnp.maximum(lmax, lmid)
    lmid_f = jnp.maximum(jnp.minimum(lmax, lmid), lmin)
    return 1.0 - lmid_f / (lmax_f + 1e-12)


def _sum25(v):
    acc = ((v[:, 0:8] + v[:, 8:16]) + v[:, 16:24]) + v[:, 24:32]
    f = acc[:, 0:4] + acc[:, 4:8]
    f = f[:, 0:2] + f[:, 2:4]
    return f[:, 0:1] + f[:, 1:2]


def _aniso_kernel(x_ref, y_ref, z_ref, xt_ref, yt_ref, zt_ref, p_ref, pt_ref,
                  e_ref):
    x, y, z = x_ref[...], y_ref[...], z_ref[...]
    xt, yt, zt = xt_ref[...], yt_ref[...], zt_ref[...]
    dot = jnp.dot(p_ref[...], pt_ref[...], preferred_element_type=jnp.float32)
    sq_i = (x * x + z * z) + y * y
    sq_j = (xt * xt + zt * zt) + yt * yt
    negd2 = -((sq_i + sq_j) - 2.0 * dot)

    keys = _key_of(negd2)
    lane = jax.lax.broadcasted_iota(jnp.int32, (RB, N), 1)
    lane128 = jax.lax.broadcasted_iota(jnp.int32, (RB, 128), 1)
    xb = jnp.broadcast_to(xt, (RB, N))
    yb = jnp.broadcast_to(yt, (RB, N))
    zb = jnp.broadcast_to(zt, (RB, N))
    ninf = jnp.float32(-jnp.inf)
    nbx = jnp.zeros((RB, 128), jnp.float32)
    nby = jnp.zeros((RB, 128), jnp.float32)
    nbz = jnp.zeros((RB, 128), jnp.float32)
    for t in range(K1):
        m = jnp.max(keys, axis=1, keepdims=True)
        sel = keys == m
        idx = jnp.min(jnp.where(sel, lane, N), axis=1, keepdims=True)
        one = lane == idx
        px = jnp.max(jnp.where(one, xb, ninf), axis=1, keepdims=True)
        py = jnp.max(jnp.where(one, yb, ninf), axis=1, keepdims=True)
        pz = jnp.max(jnp.where(one, zb, ninf), axis=1, keepdims=True)
        tm = lane128 == t
        nbx = jnp.where(tm, px, nbx)
        nby = jnp.where(tm, py, nby)
        nbz = jnp.where(tm, pz, nbz)
        keys = jnp.where(one, IMIN, keys)

    s04 = np.float32(0.04)
    mux = _sum25(nbx) * s04
    muy = _sum25(nby) * s04
    muz = _sum25(nbz) * s04
    valid = lane128 < K1
    cx = jnp.where(valid, nbx - mux, 0.0)
    cy = jnp.where(valid, nby - muy, 0.0)
    cz = jnp.where(valid, nbz - muz, 0.0)
    cx = cx.astype(jnp.bfloat16).astype(jnp.float32)
    cy = cy.astype(jnp.bfloat16).astype(jnp.float32)
    cz = cz.astype(jnp.bfloat16).astype(jnp.float32)
    cxx = jnp.sum(cx * cx, axis=1, keepdims=True) * s04
    cxy = jnp.sum(cx * cy, axis=1, keepdims=True) * s04
    cxz = jnp.sum(cx * cz, axis=1, keepdims=True) * s04
    cyy = jnp.sum(cy * cy, axis=1, keepdims=True) * s04
    cyz = jnp.sum(cy * cz, axis=1, keepdims=True) * s04
    czz = jnp.sum(cz * cz, axis=1, keepdims=True) * s04
    del s04
    e_ref[...] = _eig_e(cxx, cxy, cxz, cyy, cyz, czz)


def _aniso(pts):
    x = pts[:, 0:1]
    y = pts[:, 1:2]
    z = pts[:, 2:3]
    col = pl.BlockSpec((RB, 1), lambda i: (i, 0))
    row = pl.BlockSpec((1, N), lambda i: (0, 0))
    return pl.pallas_call(
        _aniso_kernel,
        grid=(N // RB,),
        in_specs=[col, col, col, row, row, row,
                  pl.BlockSpec((RB, 3), lambda i: (i, 0)),
                  pl.BlockSpec((3, N), lambda i: (0, 0))],
        out_specs=col,
        out_shape=jax.ShapeDtypeStruct((N, 1), jnp.float32),
        compiler_params=pltpu.CompilerParams(
            dimension_semantics=("parallel",)),
    )(x, y, z, x.T, y.T, z.T, pts, pts.T)


def _aff_kernel(a_ref, bt_ref, aff_ref, afft_ref):
    aff = jnp.dot(a_ref[...], bt_ref[...],
                  preferred_element_type=jnp.float32) * INV_SQRT_D
    aff_ref[...] = aff
    afft_ref[...] = aff.T


def _aff(s_feat, r_featT):
    return pl.pallas_call(
        _aff_kernel,
        grid=(N // RB,),
        in_specs=[pl.BlockSpec((RB, D), lambda i: (i, 0)),
                  pl.BlockSpec((D, N), lambda i: (0, 0))],
        out_specs=(pl.BlockSpec((RB, N), lambda i: (i, 0)),
                   pl.BlockSpec((N, RB), lambda i: (0, i))),
        out_shape=(jax.ShapeDtypeStruct((N, N), jnp.float32),
                   jax.ShapeDtypeStruct((N, N), jnp.float32)),
        compiler_params=pltpu.CompilerParams(
            dimension_semantics=("parallel",)),
    )(s_feat, r_featT)


def _colstats_kernel(a_ref, mx_ref, sm_ref):
    a = a_ref[...]
    mx = jnp.max(a, axis=0, keepdims=True)
    ex = jnp.exp(a - mx)
    sm_ref[...] = jnp.sum(ex, axis=0, keepdims=True)
    mx_ref[...] = mx


def _colstats(aff):
    CB = 256
    return pl.pallas_call(
        _colstats_kernel,
        grid=(N // CB,),
        in_specs=[pl.BlockSpec((N, CB), lambda i: (0, i))],
        out_specs=(pl.BlockSpec((1, CB), lambda i: (0, i)),
                   pl.BlockSpec((1, CB), lambda i: (0, i))),
        out_shape=(jax.ShapeDtypeStruct((1, N), jnp.float32),
                   jax.ShapeDtypeStruct((1, N), jnp.float32)),
        compiler_params=pltpu.CompilerParams(
            dimension_semantics=("parallel",)),
    )(aff)


def _score_kernel(a_ref, cmx_ref, csm_ref, rmx_ref, rsm_ref, es_ref, er_ref,
                  k_ref, rm_ref):
    a = a_ref[...]
    e0 = jnp.exp(a - cmx_ref[...])
    d13 = e0 / csm_ref[...]
    e1 = jnp.exp(a - rmx_ref[...])
    d12 = e1 / rsm_ref[...]
    sc = ((d13 * d12) * es_ref[...]) * er_ref[...]
    k = _key_of(sc)
    k_ref[...] = k
    rm_ref[...] = jnp.max(k, axis=1, keepdims=True)


def _scores(aff, cmx, csm, rmx, rsm, e_src, e_ref):
    col = pl.BlockSpec((RB, 1), lambda i: (i, 0))
    row = pl.BlockSpec((1, N), lambda i: (0, 0))
    return pl.pallas_call(
        _score_kernel,
        grid=(N // RB,),
        in_specs=[pl.BlockSpec((RB, N), lambda i: (i, 0)),
                  row, row, col, col, col, row],
        out_specs=(pl.BlockSpec((RB, N), lambda i: (i, 0)),
                   pl.BlockSpec((RB, 1), lambda i: (i, 0))),
        out_shape=(jax.ShapeDtypeStruct((N, N), jnp.int32),
                   jax.ShapeDtypeStruct((N, 1), jnp.int32)),
        compiler_params=pltpu.CompilerParams(
            dimension_semantics=("parallel",)),
    )(aff, cmx, csm, rmx, rsm, e_src, e_ref)


def _topk_kernel(k_ref, rm_ref, v_ref, s_ref, r_ref, keys):
    keys[...] = k_ref[...]
    cio = jax.lax.broadcasted_iota(jnp.int32, (1, N), 1)
    tio = jax.lax.broadcasted_iota(jnp.int32, (1, BW), 1)
    fio = (jax.lax.broadcasted_iota(jnp.int32, (16, 128), 0) * 128
           + jax.lax.broadcasted_iota(jnp.int32, (16, 128), 1))
    rm0 = rm_ref[...]

    def body(t, carry):
        vacc, sacc, racc, rm = carry
        m = jnp.max(rm)
        r = jnp.min(jnp.where(rm == m, fio, N))
        rowk = keys[pl.ds(r, 1), :]
        c = jnp.min(jnp.where(rowk == m, cio, N))
        here = tio == t
        vacc = jnp.where(here, _val_of(m), vacc)
        sacc = jnp.where(here, r, sacc)
        racc = jnp.where(here, c, racc)
        newrow = jnp.where(cio == c, IMIN, rowk)
        keys[pl.ds(r, 1), :] = newrow
        rm = jnp.where(fio == r, jnp.max(newrow), rm)
        return vacc, sacc, racc, rm

    v0 = jnp.zeros((1, BW), jnp.float32)
    i0 = jnp.zeros((1, BW), jnp.int32)
    v, s, r, _ = jax.lax.fori_loop(0, BW, body, (v0, i0, i0, rm0))
    v_ref[...] = v
    s_ref[...] = s
    r_ref[...] = r


def _topk(keysmat, rowmax16):
    return pl.pallas_call(
        _topk_kernel,
        out_shape=(jax.ShapeDtypeStruct((1, BW), jnp.float32),
                   jax.ShapeDtypeStruct((1, BW), jnp.int32),
                   jax.ShapeDtypeStruct((1, BW), jnp.int32)),
        scratch_shapes=[pltpu.VMEM((N, N), jnp.int32)],
    )(keysmat, rowmax16)


def kernel(src_points_f, ref_points_f, s_n_features, r_n_features,
           gt_transform, src_points, ref_points):
    e_src = _aniso(src_points_f)
    e_ref = _aniso(ref_points_f)
    aff, afft = _aff(s_n_features, r_n_features.T)
    cmx, csm = _colstats(aff)
    rmx_t, rsm_t = _colstats(afft)
    keys, rowmax = _scores(aff, cmx, csm, rmx_t.T, rsm_t.T, e_src, e_ref.T)
    vals, src, ref = _topk(keys, rowmax.reshape(16, 128))
    return vals.reshape(BW), src.reshape(BW), ref.reshape(BW)
